# baseline (speedup 1.0000x reference)
import functools

import jax
import jax.numpy as jnp
from jax import lax
from jax.experimental import pallas as pl
from jax.experimental.pallas import tpu as pltpu
from jax.experimental.pallas import tpu_sc as plsc

VOCAB = 1000000
EMB = 64
B = 16384
L = 50
HID = 128

NC = 2
NS = 16
NW = NC * NS
BPW = B // NW
GCH = 128
NK = BPW // GCH

VBLK = 32768
NGRID = (VOCAB + VBLK - 1) // VBLK
HSH = VBLK.bit_length() - 2

P = 4
PHASE_NB = [NGRID // P + (1 if p < NGRID % P else 0) for p in range(P)]
PHASE_START = [sum(PHASE_NB[:p]) for p in range(P)]


def _tc_detile_phase(tabT, start, nb):
    in_top = min(start + nb - 1, NGRID - 1)

    def body(t_ref, o_ref):
        i = pl.program_id(0)

        @pl.when(i < nb)
        def _():
            t = t_ref[...].T
            y = jnp.concatenate([t[: VBLK // 2, :], t[VBLK // 2 :, :]], axis=1)
            o_ref[...] = y.reshape(VBLK * EMB)

        @pl.when(i == nb)
        def _():
            o_ref[...] = jnp.zeros((VBLK * EMB,), jnp.float32)

    return pl.pallas_call(
        body,
        grid=(nb + 1,),
        in_specs=[
            pl.BlockSpec(
                (EMB, VBLK),
                lambda i: (0, jnp.minimum(start + i, in_top)),
            )
        ],
        out_specs=pl.BlockSpec((VBLK * EMB,), lambda i: (i,)),
        out_shape=jax.ShapeDtypeStruct(((nb + 1) * VBLK * EMB,), jnp.float32),
    )(tabT)


def _sc_bag_partial(x_flat, tab_p, s_row, n_row):
    mesh = plsc.VectorSubcoreMesh(core_axis_name="c", subcore_axis_name="s")

    @functools.partial(
        pl.kernel,
        out_type=jax.ShapeDtypeStruct((B, EMB), jnp.float32),
        mesh=mesh,
        name="bag_partial",
        compiler_params=pltpu.CompilerParams(
            use_tc_tiling_on_sc=False, needs_layout_passes=False
        ),
        scratch_types=[
            pltpu.VMEM((BPW * L,), jnp.int32),
            pltpu.VMEM((L, NK, GCH), jnp.int32),
            pltpu.VMEM((BPW, EMB), jnp.float32),
            pltpu.SemaphoreType.DMA,
        ],
    )
    def kern(x_hbm, tab_hbm, out_hbm, raw_v, idx_v, acc_v, sem):
        wid = lax.axis_index("s") * NC + lax.axis_index("c")
        pltpu.sync_copy(x_hbm.at[pl.ds(wid * (BPW * L), BPW * L)], raw_v)

        lane = lax.iota(jnp.int32, 16) * L

        def transpose_row(j):
            for kk in range(NK):
                for g in range(GCH // 16):
                    v = plsc.load_gather(
                        raw_v, [lane + (kk * GCH + g * 16) * L + j]
                    )
                    pv = (
                        (v & jnp.int32(~(VBLK - 1)))
                        + ((v & jnp.int32(VBLK // 2 - 1)) << 1)
                        + ((v >> HSH) & jnp.int32(1))
                    ) - s_row
                    pv = jnp.where(
                        (pv >= 0) & (pv < n_row), pv, jnp.int32(n_row)
                    )
                    idx_v[j, kk, pl.ds(g * 16, 16)] = pv

        transpose_row(0)
        cps = [
            pltpu.async_copy(
                tab_hbm.at[idx_v.at[0, kk]],
                acc_v.at[pl.ds(kk * GCH, GCH)],
                sem,
            )
            for kk in range(NK)
        ]
        for cp in cps:
            cp.wait()

        @pl.loop(1, L)
        def _(j):
            transpose_row(j)
            for kk in range(NK):
                pltpu.async_copy(
                    tab_hbm.at[idx_v.at[j, kk]],
                    acc_v.at[pl.ds(kk * GCH, GCH)],
                    sem,
                    add=True,
                )

        @pl.loop(1, L)
        def _(j):
            for kk in range(NK):
                pltpu.make_async_copy(
                    tab_hbm.at[idx_v.at[0, kk]],
                    acc_v.at[pl.ds(kk * GCH, GCH)],
                    sem,
                ).wait()

        pltpu.sync_copy(acc_v, out_hbm.at[pl.ds(wid * BPW, BPW)])

    return kern(x_flat, tab_p)


def _tc_mlp(partials, W1, b1, W2, b2):
    BLK = 2048

    def body(*refs):
        s_refs = refs[:P]
        w1_ref, b1_ref, w2_ref, b2_ref, o_ref = refs[P:]
        e = s_refs[0][...]
        for s_ref in s_refs[1:]:
            e = e + s_ref[...]
        e = e * (1.0 / L)
        h = jnp.dot(e, w1_ref[...], preferred_element_type=jnp.float32)
        h = jnp.maximum(h + b1_ref[...], 0.0)
        p = jnp.dot(h, w2_ref[...], preferred_element_type=jnp.float32)
        o_ref[...] = jax.nn.sigmoid(p + b2_ref[...])

    return pl.pallas_call(
        body,
        grid=(B // BLK,),
        in_specs=[pl.BlockSpec((BLK, EMB), lambda i: (i, 0)) for _ in range(P)]
        + [
            pl.BlockSpec((EMB, HID), lambda i: (0, 0)),
            pl.BlockSpec((1, HID), lambda i: (0, 0)),
            pl.BlockSpec((HID, 1), lambda i: (0, 0)),
            pl.BlockSpec((1, 1), lambda i: (0, 0)),
        ],
        out_specs=pl.BlockSpec((BLK, 1), lambda i: (i, 0)),
        out_shape=jax.ShapeDtypeStruct((B, 1), jnp.float32),
    )(*partials, W1, b1, W2, b2)


def kernel(x, emb_table, W1, b1, W2, b2):
    tabT = emb_table.T
    x_flat = x.reshape(B * L)
    partials = []
    for p in range(P):
        nb = PHASE_NB[p]
        tab_p = _tc_detile_phase(tabT, PHASE_START[p], nb)
        tab_p = tab_p.reshape((nb + 1) * VBLK, EMB)
        partials.append(
            _sc_bag_partial(x_flat, tab_p, PHASE_START[p] * VBLK, nb * VBLK)
        )
    return _tc_mlp(partials, W1, b1.reshape(1, HID), W2, b2.reshape(1, 1))


# speedup vs baseline: 118.0058x; 118.0058x over previous
import functools

import jax
import jax.numpy as jnp
from jax import lax
from jax.experimental import pallas as pl
from jax.experimental.pallas import tpu as pltpu
from jax.experimental.pallas import tpu_sc as plsc

VOCAB = 1000000
EMB = 64
B = 16384
L = 50
HID = 128

NC = 2
NS = 16
NW = NC * NS
BPW = B // NW
GCH = 128
NK = BPW // GCH

VBLK = 32768
NGRID = (VOCAB + VBLK - 1) // VBLK
HSH = VBLK.bit_length() - 2

P = 4
PHASE_NB = [NGRID // P + (1 if p < NGRID % P else 0) for p in range(P)]
PHASE_START = [sum(PHASE_NB[:p]) for p in range(P)]


def _tc_detile_phase(tabT, start, nb):
    def body(t_ref, o_ref):
        t = t_ref[...].T
        y = jnp.concatenate([t[: VBLK // 2, :], t[VBLK // 2 :, :]], axis=1)
        o_ref[...] = y.reshape(VBLK * EMB)

    return pl.pallas_call(
        body,
        grid=(nb,),
        in_specs=[pl.BlockSpec((EMB, VBLK), lambda i: (0, start + i))],
        out_specs=pl.BlockSpec((VBLK * EMB,), lambda i: (i,)),
        out_shape=jax.ShapeDtypeStruct((nb * VBLK * EMB,), jnp.float32),
    )(tabT)


def _sc_bag_partial(x_flat, tab_p, s_row, n_row):
    mesh = plsc.VectorSubcoreMesh(core_axis_name="c", subcore_axis_name="s")

    @functools.partial(
        pl.kernel,
        out_type=jax.ShapeDtypeStruct((B, EMB), jnp.float32),
        mesh=mesh,
        name="bag_partial",
        compiler_params=pltpu.CompilerParams(
            use_tc_tiling_on_sc=False, needs_layout_passes=False
        ),
        scratch_types=[
            pltpu.VMEM((BPW * L,), jnp.int32),
            pltpu.VMEM((L, NK, GCH), jnp.int32),
            pltpu.VMEM((BPW, EMB), jnp.float32),
            pltpu.SemaphoreType.DMA,
        ],
    )
    def kern(x_hbm, tab_hbm, out_hbm, raw_v, idx_v, acc_v, sem):
        wid = lax.axis_index("s") * NC + lax.axis_index("c")
        pltpu.sync_copy(x_hbm.at[pl.ds(wid * (BPW * L), BPW * L)], raw_v)

        lane = lax.iota(jnp.int32, 16) * L

        def transpose_row(j):
            for kk in range(NK):
                for g in range(GCH // 16):
                    v = plsc.load_gather(
                        raw_v, [lane + (kk * GCH + g * 16) * L + j]
                    )
                    pv = (
                        (v & jnp.int32(~(VBLK - 1)))
                        + ((v & jnp.int32(VBLK // 2 - 1)) << 1)
                        + ((v >> HSH) & jnp.int32(1))
                    ) - s_row
                    pv = jnp.where(
                        (pv >= 0) & (pv < n_row), pv, jnp.int32(-1)
                    )
                    idx_v[j, kk, pl.ds(g * 16, 16)] = pv

        zeros16 = jnp.zeros((16,), jnp.float32)

        @pl.loop(0, BPW)
        def _(i):
            for e in range(EMB // 16):
                acc_v[i, pl.ds(e * 16, 16)] = zeros16

        @pl.loop(0, L)
        def _(j):
            transpose_row(j)
            for kk in range(NK):
                pltpu.async_copy(
                    tab_hbm.at[
                        plsc.Indices(idx_v.at[j, kk], ignored_value=-1)
                    ],
                    acc_v.at[pl.ds(kk * GCH, GCH)],
                    sem,
                    add=True,
                )

        @pl.loop(0, L)
        def _(j):
            for kk in range(NK):
                pltpu.make_async_copy(
                    tab_hbm.at[
                        plsc.Indices(idx_v.at[j, kk], ignored_value=-1)
                    ],
                    acc_v.at[pl.ds(kk * GCH, GCH)],
                    sem,
                ).wait()

        pltpu.sync_copy(acc_v, out_hbm.at[pl.ds(wid * BPW, BPW)])

    return kern(x_flat, tab_p)


def _tc_mlp(partials, W1, b1, W2, b2):
    BLK = 2048

    def body(*refs):
        s_refs = refs[:P]
        w1_ref, b1_ref, w2_ref, b2_ref, o_ref = refs[P:]
        e = s_refs[0][...]
        for s_ref in s_refs[1:]:
            e = e + s_ref[...]
        e = e * (1.0 / L)
        h = jnp.dot(e, w1_ref[...], preferred_element_type=jnp.float32)
        h = jnp.maximum(h + b1_ref[...], 0.0)
        p = jnp.dot(h, w2_ref[...], preferred_element_type=jnp.float32)
        o_ref[...] = jax.nn.sigmoid(p + b2_ref[...])

    return pl.pallas_call(
        body,
        grid=(B // BLK,),
        in_specs=[pl.BlockSpec((BLK, EMB), lambda i: (i, 0)) for _ in range(P)]
        + [
            pl.BlockSpec((EMB, HID), lambda i: (0, 0)),
            pl.BlockSpec((1, HID), lambda i: (0, 0)),
            pl.BlockSpec((HID, 1), lambda i: (0, 0)),
            pl.BlockSpec((1, 1), lambda i: (0, 0)),
        ],
        out_specs=pl.BlockSpec((BLK, 1), lambda i: (i, 0)),
        out_shape=jax.ShapeDtypeStruct((B, 1), jnp.float32),
    )(*partials, W1, b1, W2, b2)


def kernel(x, emb_table, W1, b1, W2, b2):
    tabT = emb_table.T
    x_flat = x.reshape(B * L)
    partials = []
    for p in range(P):
        nb = PHASE_NB[p]
        tab_p = _tc_detile_phase(tabT, PHASE_START[p], nb)
        tab_p = tab_p.reshape(nb * VBLK, EMB)
        partials.append(
            _sc_bag_partial(x_flat, tab_p, PHASE_START[p] * VBLK, nb * VBLK)
        )
    return _tc_mlp(partials, W1, b1.reshape(1, HID), W2, b2.reshape(1, 1))
